# baseline (speedup 1.0000x reference)
import functools

import jax
import jax.numpy as jnp
from jax import lax
from jax.experimental import pallas as pl
from jax.experimental.pallas import tpu as pltpu
from jax.experimental.pallas import tpu_sc as plsc

EMBED = 128
BATCH = 16384
NUM_CORES = 2
NUM_SUBCORES = 16
NUM_WORKERS = NUM_CORES * NUM_SUBCORES
ROWS_PER_WORKER = BATCH // NUM_WORKERS
CHUNK = 128
NUM_CHUNKS = ROWS_PER_WORKER // CHUNK


def _gather_body(ids_hbm, tables_hbm, out_hbm, idx_v, rows_v, tab_sh, gsem, wsem):
    sid = lax.axis_index("s")
    wid = sid * NUM_CORES + lax.axis_index("c")
    base = wid * ROWS_PER_WORKER
    with jax.named_scope("stage"):
        pltpu.async_copy(tables_hbm, tab_sh, gsem)
        pltpu.async_copy(ids_hbm.at[pl.ds(wid * NUM_CHUNKS, NUM_CHUNKS)], idx_v, gsem)
        pltpu.make_async_copy(tables_hbm, tab_sh, gsem).wait()
        pltpu.make_async_copy(
            ids_hbm.at[pl.ds(wid * NUM_CHUNKS, NUM_CHUNKS)], idx_v, gsem
        ).wait()
    with jax.named_scope("gather"):
        for c in range(NUM_CHUNKS):
            pltpu.async_copy(
                tab_sh.at[idx_v.at[c]], rows_v.at[pl.ds(c * CHUNK, CHUNK)], gsem
            )
        for c in range(NUM_CHUNKS):
            pltpu.make_async_copy(
                tab_sh.at[idx_v.at[c]], rows_v.at[pl.ds(c * CHUNK, CHUNK)], gsem
            ).wait()
            pltpu.async_copy(
                rows_v.at[pl.ds(c * CHUNK, CHUNK)],
                out_hbm.at[pl.ds(base + c * CHUNK, CHUNK)],
                wsem,
            )
    with jax.named_scope("write"):
        for c in range(NUM_CHUNKS):
            pltpu.make_async_copy(
                rows_v.at[pl.ds(c * CHUNK, CHUNK)],
                out_hbm.at[pl.ds(base + c * CHUNK, CHUNK)],
                wsem,
            ).wait()


@jax.jit
def _run(ids2d, tables):
    mesh = plsc.VectorSubcoreMesh(core_axis_name="c", subcore_axis_name="s")
    f = pl.kernel(
        _gather_body,
        mesh=mesh,
        out_type=jax.ShapeDtypeStruct((BATCH, EMBED), jnp.float32),
        scratch_types=[
            pltpu.VMEM((NUM_CHUNKS, CHUNK), jnp.int32),
            pltpu.VMEM((ROWS_PER_WORKER, EMBED), jnp.float32),
            pltpu.VMEM_SHARED((6, EMBED), jnp.float32),
            pltpu.SemaphoreType.DMA,
            pltpu.SemaphoreType.DMA,
        ],
    )
    return f(ids2d, tables)


def kernel(dataset_ids, tables):
    ids2d = dataset_ids.astype(jnp.int32).reshape(BATCH // CHUNK, CHUNK)
    return _run(ids2d, tables)


# speedup vs baseline: 1.9368x; 1.9368x over previous
import functools

import jax
import jax.numpy as jnp
from jax import lax
from jax.experimental import pallas as pl
from jax.experimental.pallas import tpu as pltpu

EMBED = 128
BATCH = 16384
BLK = 1024
GRID = BATCH // BLK


def _tc_body(ids_ref, tab_ref, out_ref):
    ids = ids_ref[0, 0, :].reshape(BLK, 1)
    onehot = (ids == lax.broadcasted_iota(jnp.int32, (BLK, 8), 1)).astype(jnp.float32)
    out_ref[...] = jnp.dot(onehot, tab_ref[...], preferred_element_type=jnp.float32)


@jax.jit
def _run_tc(ids3d, tab8):
    return pl.pallas_call(
        _tc_body,
        grid=(GRID,),
        in_specs=[
            pl.BlockSpec((1, 1, BLK), lambda i: (i, 0, 0)),
            pl.BlockSpec((8, EMBED), lambda i: (0, 0)),
        ],
        out_specs=pl.BlockSpec((BLK, EMBED), lambda i: (i, 0)),
        out_shape=jax.ShapeDtypeStruct((BATCH, EMBED), jnp.float32),
    )(ids3d, tab8)


def kernel(dataset_ids, tables):
    ids3d = dataset_ids.astype(jnp.int32).reshape(GRID, 1, BLK)
    tab8 = jnp.pad(tables, ((0, 2), (0, 0)))
    return _run_tc(ids3d, tab8)
